# baseline (speedup 1.0000x reference)
import functools

import jax
import jax.numpy as jnp
from jax import lax
from jax.experimental import pallas as pl
from jax.experimental.pallas import tpu as pltpu
from jax.experimental.pallas import tpu_sc as plsc

N = 10000
E = 320000
D = 128

NC = 2
NS = 16
L = 16
NW = NC * NS

E_PER_TILE = E // NW
CHUNK = 80
NCHUNK = E_PER_TILE // CHUNK
STRIPE = 624
REM = N - NS * STRIPE

_mesh = plsc.VectorSubcoreMesh(core_axis_name="c", subcore_axis_name="s")


def _wid_base(c, s):
    return c * (E // NC) + s * E_PER_TILE


def _zero_stripe(acc_sh, rb, c, s):
    def _zero_row(r, _):
        for j in range(D // 16):
            rb[r, pl.ds(j * 16, 16)] = jnp.zeros((16,), jnp.float32)
        return 0

    lax.fori_loop(0, CHUNK, _zero_row, 0)
    stripe = s * STRIPE
    for k in range(7):
        pltpu.sync_copy(rb, acc_sh.at[pl.ds(stripe + k * CHUNK, CHUNK)])
    pltpu.sync_copy(rb.at[pl.ds(0, 64)], acc_sh.at[pl.ds(stripe + 560, 64)])

    @pl.when(s == NS - 1)
    def _zero_rem():
        pltpu.sync_copy(rb.at[pl.ds(0, REM)], acc_sh.at[pl.ds(NS * STRIPE, REM)])


def _copy_out(acc_sh, out_hbm, rb, c, s):
    stripe = s * STRIPE
    for k in range(7):
        pltpu.sync_copy(acc_sh.at[pl.ds(stripe + k * CHUNK, CHUNK)], rb)
        pltpu.sync_copy(rb, out_hbm.at[pl.ds(c * N + stripe + k * CHUNK, CHUNK)])
    pltpu.sync_copy(acc_sh.at[pl.ds(stripe + 560, 64)], rb.at[pl.ds(0, 64)])
    pltpu.sync_copy(rb.at[pl.ds(0, 64)],
                    out_hbm.at[pl.ds(c * N + stripe + 560, 64)])

    @pl.when(s == NS - 1)
    def _copy_rem():
        pltpu.sync_copy(acc_sh.at[pl.ds(NS * STRIPE, REM)], rb.at[pl.ds(0, REM)])
        pltpu.sync_copy(rb.at[pl.ds(0, REM)],
                        out_hbm.at[pl.ds(c * N + NS * STRIPE, REM)])


@functools.partial(
    pl.kernel,
    mesh=_mesh,
    out_type=jax.ShapeDtypeStruct((NC * N, D), jnp.float32),
    scratch_types=[
        pltpu.VMEM((E_PER_TILE,), jnp.int32),
        pltpu.VMEM((E_PER_TILE,), jnp.float32),
        pltpu.VMEM((CHUNK,), jnp.int32),
        pltpu.VMEM((CHUNK,), jnp.int32),
        pltpu.VMEM((CHUNK, D), jnp.float32),
        pltpu.VMEM((CHUNK, D), jnp.float32),
        pltpu.VMEM_SHARED((N, D), jnp.float32),
        pltpu.SemaphoreType.DMA,
    ],
)
def _deg_kernel(dst_hbm, ew_hbm, out_hbm, dst_all, ew_all,
                db0, db1, rb0, rb1, acc_sh, ssem):
    c = lax.axis_index("c")
    s = lax.axis_index("s")
    base = _wid_base(c, s)
    dbs = (db0, db1)
    rbs = (rb0, rb1)

    pltpu.sync_copy(dst_hbm.at[pl.ds(base, E_PER_TILE)], dst_all)
    pltpu.sync_copy(ew_hbm.at[pl.ds(base, E_PER_TILE)], ew_all)
    _zero_stripe(acc_sh, rb0, c, s)
    plsc.subcore_barrier()

    def _drain_s():
        pltpu.make_async_copy(out_hbm.at[pl.ds(0, CHUNK)], rb0, ssem).wait()

    def _build_and_scatter(i, db, rb):
        for g in range(CHUNK // 16):
            db[pl.ds(g * 16, 16)] = dst_all[pl.ds(i * CHUNK + g * 16, 16)]

        def _group(g, _):
            wv = ew_all[pl.ds(i * CHUNK + g * 16, 16)]
            for k in range(16):
                w = jnp.full((16,), wv[k], jnp.float32)
                for j in range(D // 16):
                    rb[g * 16 + k, pl.ds(j * 16, 16)] = w
            return 0

        lax.fori_loop(0, CHUNK // 16, _group, 0)
        pltpu.async_copy(rb, acc_sh.at[db], ssem, add=True)

    def _pair(sup, _):
        for b in range(2):
            i = sup * 2 + b

            @pl.when(i >= 2)
            def _():
                _drain_s()

            _build_and_scatter(i, dbs[b], rbs[b])
        return 0

    lax.fori_loop(0, NCHUNK // 2, _pair, 0)
    _drain_s()
    _drain_s()
    _build_and_scatter(NCHUNK - 1, db0, rb0)
    _drain_s()
    plsc.subcore_barrier()
    _copy_out(acc_sh, out_hbm, rb1, c, s)


@functools.partial(
    pl.kernel,
    mesh=_mesh,
    out_type=jax.ShapeDtypeStruct((NC * N, D), jnp.float32),
    scratch_types=[
        pltpu.VMEM((CHUNK,), jnp.int32),
        pltpu.VMEM((CHUNK,), jnp.int32),
        pltpu.VMEM((CHUNK,), jnp.int32),
        pltpu.VMEM((CHUNK,), jnp.int32),
        pltpu.VMEM((CHUNK,), jnp.int32),
        pltpu.VMEM((CHUNK,), jnp.int32),
        pltpu.VMEM((CHUNK,), jnp.int32),
        pltpu.VMEM((CHUNK,), jnp.int32),
        pltpu.VMEM((CHUNK,), jnp.float32),
        pltpu.VMEM((CHUNK,), jnp.float32),
        pltpu.VMEM((CHUNK,), jnp.float32),
        pltpu.VMEM((CHUNK,), jnp.float32),
        pltpu.VMEM((CHUNK, D), jnp.float32),
        pltpu.VMEM((CHUNK, D), jnp.float32),
        pltpu.VMEM((CHUNK, D), jnp.float32),
        pltpu.VMEM((CHUNK, D), jnp.float32),
        pltpu.VMEM_SHARED((N, D), jnp.float32),
        pltpu.SemaphoreType.DMA,
        pltpu.SemaphoreType.DMA,
        pltpu.SemaphoreType.DMA,
    ],
)
def _edge_kernel(xw_hbm, src_hbm, dst_hbm, ew_hbm, out_hbm,
                 sb0, sb1, sb2, sb3, db0, db1, db2, db3,
                 eb0, eb1, eb2, eb3, rb0, rb1, rb2, rb3,
                 acc_sh, isem, gsem, ssem):
    c = lax.axis_index("c")
    s = lax.axis_index("s")
    base = _wid_base(c, s)
    sbs = (sb0, sb1, sb2, sb3)
    dbs = (db0, db1, db2, db3)
    ebs = (eb0, eb1, eb2, eb3)
    rbs = (rb0, rb1, rb2, rb3)

    def _istart(i, m):
        off = base + i * CHUNK
        pltpu.async_copy(src_hbm.at[pl.ds(off, CHUNK)], sbs[m], isem)
        pltpu.async_copy(dst_hbm.at[pl.ds(off, CHUNK)], dbs[m], isem)
        pltpu.async_copy(ew_hbm.at[pl.ds(off, CHUNK)], ebs[m], isem)

    def _iwait():
        for _ in range(3):
            pltpu.make_async_copy(src_hbm.at[pl.ds(0, CHUNK)], sb0, isem).wait()

    def _gstart(b):
        pltpu.async_copy(xw_hbm.at[sbs[b]], rbs[b], gsem)

    def _gwait():
        pltpu.make_async_copy(xw_hbm.at[pl.ds(0, CHUNK)], rb0, gsem).wait()

    def _swait():
        pltpu.make_async_copy(xw_hbm.at[pl.ds(0, CHUNK)], rb0, ssem).wait()

    for i in range(3):
        off = base + i * CHUNK
        pltpu.sync_copy(src_hbm.at[pl.ds(off, CHUNK)], sbs[i])
        pltpu.sync_copy(dst_hbm.at[pl.ds(off, CHUNK)], dbs[i])
        pltpu.sync_copy(ew_hbm.at[pl.ds(off, CHUNK)], ebs[i])
    _zero_stripe(acc_sh, rb3, c, s)
    _gstart(0)
    _gstart(1)
    plsc.subcore_barrier()

    def _process(i, b):
        def _group(g, _):
            wv = ebs[b][pl.ds(g * 16, 16)]
            for k in range(16):
                w = jnp.full((16,), wv[k], jnp.float32)
                r = g * 16 + k
                for j in range(D // 16):
                    sl = pl.ds(j * 16, 16)
                    rbs[b][r, sl] = rbs[b][r, sl] * w
            return 0

        lax.fori_loop(0, CHUNK // 16, _group, 0)
        pltpu.async_copy(rbs[b], acc_sh.at[dbs[b]], ssem, add=True)

    def _quad(sup, _):
        for b in range(4):
            i = sup * 4 + b

            @pl.when(i >= 1)
            def _():
                _swait()

            @pl.when(i + 3 <= NCHUNK - 1)
            def _():
                _istart(i + 3, (b + 3) % 4)

            @pl.when(i + 2 <= NCHUNK - 1)
            def _():
                @pl.when(i >= 1)
                def _():
                    _iwait()

                _gstart((b + 2) % 4)

            _gwait()
            _process(i, b)
        return 0

    lax.fori_loop(0, NCHUNK // 4, _quad, 0)
    _swait()
    _gwait()
    _process(NCHUNK - 1, 0)
    _swait()
    plsc.subcore_barrier()
    _copy_out(acc_sh, out_hbm, rb1, c, s)


ROWS_TC = 400
GRID = N // ROWS_TC


def _dis_block(d0, d1):
    deg = 1.0 + d0[:, 0:1] + d1[:, 0:1]
    return jnp.where(deg > 0, lax.rsqrt(jnp.maximum(deg, 1e-12)), 0.0)


def _m1_body(x_ref, w_ref, d0_ref, d1_ref, o_ref):
    dis = _dis_block(d0_ref[...], d1_ref[...])
    xw = jnp.dot(x_ref[...], w_ref[...], preferred_element_type=jnp.float32,
                 precision=lax.Precision.HIGHEST)
    o_ref[...] = xw * dis


def _fm_body(a0_ref, a1_ref, xwp_ref, d0_ref, d1_ref, b_ref, w_ref, o_ref):
    dis = _dis_block(d0_ref[...], d1_ref[...])
    h = dis * (a0_ref[...] + a1_ref[...] + xwp_ref[...]) + b_ref[...]
    h = jnp.maximum(h, 0.0)
    hw = jnp.dot(h, w_ref[...], preferred_element_type=jnp.float32,
                 precision=lax.Precision.HIGHEST)
    o_ref[...] = hw * dis


def _f2_body(a0_ref, a1_ref, xwp_ref, d0_ref, d1_ref, b_ref, o_ref):
    dis = _dis_block(d0_ref[...], d1_ref[...])
    o_ref[...] = dis * (a0_ref[...] + a1_ref[...] + xwp_ref[...]) + b_ref[...]


_row_spec = pl.BlockSpec((ROWS_TC, D), lambda i: (i, 0))
_deg_spec = pl.BlockSpec((ROWS_TC, 8), lambda i: (i, 0))
_w_spec = pl.BlockSpec((D, D), lambda i: (0, 0))
_b_spec = pl.BlockSpec((1, D), lambda i: (0, 0))
_out_struct = jax.ShapeDtypeStruct((N, D), jnp.float32)

_m1 = pl.pallas_call(
    _m1_body, grid=(GRID,),
    in_specs=[_row_spec, _w_spec, _deg_spec, _deg_spec],
    out_specs=_row_spec, out_shape=_out_struct)

_fm = pl.pallas_call(
    _fm_body, grid=(GRID,),
    in_specs=[_row_spec, _row_spec, _row_spec, _deg_spec, _deg_spec,
              _b_spec, _w_spec],
    out_specs=_row_spec, out_shape=_out_struct)

_f2 = pl.pallas_call(
    _f2_body, grid=(GRID,),
    in_specs=[_row_spec, _row_spec, _row_spec, _deg_spec, _deg_spec, _b_spec],
    out_specs=_row_spec, out_shape=_out_struct)


def kernel(x, edge_index, edge_weight, W1, b1, W2, b2):
    src = edge_index[0]
    dst = edge_index[1]

    deg_parts = _deg_kernel(dst, edge_weight)
    d0, d1 = deg_parts[:N, :8], deg_parts[N:, :8]

    xw1p = _m1(x, W1, d0, d1)
    a1 = _edge_kernel(xw1p, src, dst, edge_weight)
    xw2p = _fm(a1[:N], a1[N:], xw1p, d0, d1, b1.reshape(1, D), W2)
    a2 = _edge_kernel(xw2p, src, dst, edge_weight)
    out = _f2(a2[:N], a2[N:], xw2p, d0, d1, b2.reshape(1, D))
    return out


# speedup vs baseline: 1.0076x; 1.0076x over previous
import functools

import jax
import jax.numpy as jnp
from jax import lax
from jax.experimental import pallas as pl
from jax.experimental.pallas import tpu as pltpu
from jax.experimental.pallas import tpu_sc as plsc

N = 10000
E = 320000
D = 128

NC = 2
NS = 16
L = 16
NW = NC * NS

E_PER_TILE = E // NW
CHUNK = 80
NCHUNK = E_PER_TILE // CHUNK
STRIPE = 624
REM = N - NS * STRIPE

_mesh = plsc.VectorSubcoreMesh(core_axis_name="c", subcore_axis_name="s")


def _wid_base(c, s):
    return c * (E // NC) + s * E_PER_TILE


def _zero_stripe(acc_sh, rb, c, s):
    def _zero_row(r, _):
        for j in range(D // 16):
            rb[r, pl.ds(j * 16, 16)] = jnp.zeros((16,), jnp.float32)
        return 0

    lax.fori_loop(0, CHUNK, _zero_row, 0)
    stripe = s * STRIPE
    for k in range(7):
        pltpu.sync_copy(rb, acc_sh.at[pl.ds(stripe + k * CHUNK, CHUNK)])
    pltpu.sync_copy(rb.at[pl.ds(0, 64)], acc_sh.at[pl.ds(stripe + 560, 64)])

    @pl.when(s == NS - 1)
    def _zero_rem():
        pltpu.sync_copy(rb.at[pl.ds(0, REM)], acc_sh.at[pl.ds(NS * STRIPE, REM)])


def _copy_out(acc_sh, out_hbm, rb, c, s):
    stripe = s * STRIPE
    for k in range(7):
        pltpu.sync_copy(acc_sh.at[pl.ds(stripe + k * CHUNK, CHUNK)], rb)
        pltpu.sync_copy(rb, out_hbm.at[pl.ds(c * N + stripe + k * CHUNK, CHUNK)])
    pltpu.sync_copy(acc_sh.at[pl.ds(stripe + 560, 64)], rb.at[pl.ds(0, 64)])
    pltpu.sync_copy(rb.at[pl.ds(0, 64)],
                    out_hbm.at[pl.ds(c * N + stripe + 560, 64)])

    @pl.when(s == NS - 1)
    def _copy_rem():
        pltpu.sync_copy(acc_sh.at[pl.ds(NS * STRIPE, REM)], rb.at[pl.ds(0, REM)])
        pltpu.sync_copy(rb.at[pl.ds(0, REM)],
                        out_hbm.at[pl.ds(c * N + NS * STRIPE, REM)])


@functools.partial(
    pl.kernel,
    mesh=_mesh,
    out_type=jax.ShapeDtypeStruct((NC * N, D), jnp.float32),
    scratch_types=[
        pltpu.VMEM((E_PER_TILE,), jnp.int32),
        pltpu.VMEM((E_PER_TILE,), jnp.float32),
        pltpu.VMEM((CHUNK,), jnp.int32),
        pltpu.VMEM((CHUNK,), jnp.int32),
        pltpu.VMEM((CHUNK, D), jnp.float32),
        pltpu.VMEM((CHUNK, D), jnp.float32),
        pltpu.VMEM_SHARED((N, D), jnp.float32),
        pltpu.SemaphoreType.DMA,
    ],
)
def _deg_kernel(dst_hbm, ew_hbm, out_hbm, dst_all, ew_all,
                db0, db1, rb0, rb1, acc_sh, ssem):
    c = lax.axis_index("c")
    s = lax.axis_index("s")
    base = _wid_base(c, s)
    dbs = (db0, db1)
    rbs = (rb0, rb1)

    pltpu.sync_copy(dst_hbm.at[pl.ds(base, E_PER_TILE)], dst_all)
    pltpu.sync_copy(ew_hbm.at[pl.ds(base, E_PER_TILE)], ew_all)
    _zero_stripe(acc_sh, rb0, c, s)
    plsc.subcore_barrier()

    def _drain_s():
        pltpu.make_async_copy(out_hbm.at[pl.ds(0, CHUNK)], rb0, ssem).wait()

    def _build_and_scatter(i, db, rb):
        for g in range(CHUNK // 16):
            db[pl.ds(g * 16, 16)] = dst_all[pl.ds(i * CHUNK + g * 16, 16)]

        def _group(g, _):
            wv = ew_all[pl.ds(i * CHUNK + g * 16, 16)]
            for k in range(16):
                w = jnp.full((16,), wv[k], jnp.float32)
                for j in range(D // 16):
                    rb[g * 16 + k, pl.ds(j * 16, 16)] = w
            return 0

        lax.fori_loop(0, CHUNK // 16, _group, 0)
        pltpu.async_copy(rb, acc_sh.at[db], ssem, add=True)

    def _pair(sup, _):
        for b in range(2):
            i = sup * 2 + b

            @pl.when(i >= 2)
            def _():
                _drain_s()

            _build_and_scatter(i, dbs[b], rbs[b])
        return 0

    lax.fori_loop(0, NCHUNK // 2, _pair, 0)
    _drain_s()
    _drain_s()
    _build_and_scatter(NCHUNK - 1, db0, rb0)
    _drain_s()
    plsc.subcore_barrier()
    _copy_out(acc_sh, out_hbm, rb1, c, s)


@functools.partial(
    pl.kernel,
    mesh=_mesh,
    out_type=jax.ShapeDtypeStruct((NC * N, D), jnp.float32),
    scratch_types=[
        pltpu.VMEM((CHUNK,), jnp.int32),
        pltpu.VMEM((CHUNK,), jnp.int32),
        pltpu.VMEM((CHUNK,), jnp.int32),
        pltpu.VMEM((CHUNK,), jnp.int32),
        pltpu.VMEM((CHUNK,), jnp.int32),
        pltpu.VMEM((CHUNK,), jnp.int32),
        pltpu.VMEM((CHUNK,), jnp.int32),
        pltpu.VMEM((CHUNK,), jnp.int32),
        pltpu.VMEM((CHUNK,), jnp.float32),
        pltpu.VMEM((CHUNK,), jnp.float32),
        pltpu.VMEM((CHUNK,), jnp.float32),
        pltpu.VMEM((CHUNK,), jnp.float32),
        pltpu.VMEM((CHUNK, D), jnp.float32),
        pltpu.VMEM((CHUNK, D), jnp.float32),
        pltpu.VMEM((CHUNK, D), jnp.float32),
        pltpu.VMEM((CHUNK, D), jnp.float32),
        pltpu.VMEM_SHARED((N, D), jnp.float32),
        pltpu.SemaphoreType.DMA,
        pltpu.SemaphoreType.DMA,
        pltpu.SemaphoreType.DMA,
    ],
)
def _edge_kernel(xw_hbm, src_hbm, dst_hbm, ew_hbm, out_hbm,
                 sb0, sb1, sb2, sb3, db0, db1, db2, db3,
                 eb0, eb1, eb2, eb3, rb0, rb1, rb2, rb3,
                 acc_sh, isem, gsem, ssem):
    c = lax.axis_index("c")
    s = lax.axis_index("s")
    base = _wid_base(c, s)
    sbs = (sb0, sb1, sb2, sb3)
    dbs = (db0, db1, db2, db3)
    ebs = (eb0, eb1, eb2, eb3)
    rbs = (rb0, rb1, rb2, rb3)

    def _istart(i, m):
        off = base + i * CHUNK
        pltpu.async_copy(src_hbm.at[pl.ds(off, CHUNK)], sbs[m], isem)
        pltpu.async_copy(dst_hbm.at[pl.ds(off, CHUNK)], dbs[m], isem)
        pltpu.async_copy(ew_hbm.at[pl.ds(off, CHUNK)], ebs[m], isem)

    def _iwait():
        for _ in range(3):
            pltpu.make_async_copy(src_hbm.at[pl.ds(0, CHUNK)], sb0, isem).wait()

    def _gstart(b):
        pltpu.async_copy(xw_hbm.at[sbs[b]], rbs[b], gsem)

    def _gwait():
        pltpu.make_async_copy(xw_hbm.at[pl.ds(0, CHUNK)], rb0, gsem).wait()

    def _swait():
        pltpu.make_async_copy(xw_hbm.at[pl.ds(0, CHUNK)], rb0, ssem).wait()

    for i in range(3):
        off = base + i * CHUNK
        pltpu.sync_copy(src_hbm.at[pl.ds(off, CHUNK)], sbs[i])
        pltpu.sync_copy(dst_hbm.at[pl.ds(off, CHUNK)], dbs[i])
        pltpu.sync_copy(ew_hbm.at[pl.ds(off, CHUNK)], ebs[i])
    _zero_stripe(acc_sh, rb3, c, s)
    _gstart(0)
    _gstart(1)
    plsc.subcore_barrier()

    def _process(i, b):
        def _group(g, _):
            wv = ebs[b][pl.ds(g * 16, 16)]
            for k in range(16):
                w = jnp.full((16,), wv[k], jnp.float32)
                r = g * 16 + k
                for j in range(D // 16):
                    sl = pl.ds(j * 16, 16)
                    rbs[b][r, sl] = rbs[b][r, sl] * w
            return 0

        lax.fori_loop(0, CHUNK // 16, _group, 0)
        pltpu.async_copy(rbs[b], acc_sh.at[dbs[b]], ssem, add=True)

    def _quad(sup, _):
        for b in range(4):
            i = sup * 4 + b

            @pl.when(i + 2 <= NCHUNK - 1)
            def _():
                @pl.when(i >= 1)
                def _():
                    _iwait()

                _gstart((b + 2) % 4)

            _gwait()

            @pl.when(i >= 1)
            def _():
                _swait()

            @pl.when(i + 3 <= NCHUNK - 1)
            def _():
                _istart(i + 3, (b + 3) % 4)

            _process(i, b)
        return 0

    lax.fori_loop(0, NCHUNK // 4, _quad, 0)
    _swait()
    _gwait()
    _process(NCHUNK - 1, 0)
    _swait()
    plsc.subcore_barrier()
    _copy_out(acc_sh, out_hbm, rb1, c, s)


ROWS_TC = 400
GRID = N // ROWS_TC


def _dis_block(d0, d1):
    deg = 1.0 + d0[:, 0:1] + d1[:, 0:1]
    return jnp.where(deg > 0, lax.rsqrt(jnp.maximum(deg, 1e-12)), 0.0)


def _m1_body(x_ref, w_ref, d0_ref, d1_ref, o_ref):
    dis = _dis_block(d0_ref[...], d1_ref[...])
    xw = jnp.dot(x_ref[...], w_ref[...], preferred_element_type=jnp.float32,
                 precision=lax.Precision.HIGHEST)
    o_ref[...] = xw * dis


def _fm_body(a0_ref, a1_ref, xwp_ref, d0_ref, d1_ref, b_ref, w_ref, o_ref):
    dis = _dis_block(d0_ref[...], d1_ref[...])
    h = dis * (a0_ref[...] + a1_ref[...] + xwp_ref[...]) + b_ref[...]
    h = jnp.maximum(h, 0.0)
    hw = jnp.dot(h, w_ref[...], preferred_element_type=jnp.float32,
                 precision=lax.Precision.HIGHEST)
    o_ref[...] = hw * dis


def _f2_body(a0_ref, a1_ref, xwp_ref, d0_ref, d1_ref, b_ref, o_ref):
    dis = _dis_block(d0_ref[...], d1_ref[...])
    o_ref[...] = dis * (a0_ref[...] + a1_ref[...] + xwp_ref[...]) + b_ref[...]


_row_spec = pl.BlockSpec((ROWS_TC, D), lambda i: (i, 0))
_deg_spec = pl.BlockSpec((ROWS_TC, 8), lambda i: (i, 0))
_w_spec = pl.BlockSpec((D, D), lambda i: (0, 0))
_b_spec = pl.BlockSpec((1, D), lambda i: (0, 0))
_out_struct = jax.ShapeDtypeStruct((N, D), jnp.float32)

_m1 = pl.pallas_call(
    _m1_body, grid=(GRID,),
    in_specs=[_row_spec, _w_spec, _deg_spec, _deg_spec],
    out_specs=_row_spec, out_shape=_out_struct)

_fm = pl.pallas_call(
    _fm_body, grid=(GRID,),
    in_specs=[_row_spec, _row_spec, _row_spec, _deg_spec, _deg_spec,
              _b_spec, _w_spec],
    out_specs=_row_spec, out_shape=_out_struct)

_f2 = pl.pallas_call(
    _f2_body, grid=(GRID,),
    in_specs=[_row_spec, _row_spec, _row_spec, _deg_spec, _deg_spec, _b_spec],
    out_specs=_row_spec, out_shape=_out_struct)


def kernel(x, edge_index, edge_weight, W1, b1, W2, b2):
    src = edge_index[0]
    dst = edge_index[1]

    deg_parts = _deg_kernel(dst, edge_weight)
    d0, d1 = deg_parts[:N, :8], deg_parts[N:, :8]

    xw1p = _m1(x, W1, d0, d1)
    a1 = _edge_kernel(xw1p, src, dst, edge_weight)
    xw2p = _fm(a1[:N], a1[N:], xw1p, d0, d1, b1.reshape(1, D), W2)
    a2 = _edge_kernel(xw2p, src, dst, edge_weight)
    out = _f2(a2[:N], a2[N:], xw2p, d0, d1, b2.reshape(1, D))
    return out


# speedup vs baseline: 1.0117x; 1.0040x over previous
import functools

import jax
import jax.numpy as jnp
from jax import lax
from jax.experimental import pallas as pl
from jax.experimental.pallas import tpu as pltpu
from jax.experimental.pallas import tpu_sc as plsc

N = 10000
E = 320000
D = 128

NC = 2
NS = 16
L = 16
NW = NC * NS

E_PER_TILE = E // NW
CHUNK = 80
NCHUNK = E_PER_TILE // CHUNK
STRIPE = 624
REM = N - NS * STRIPE

_mesh = plsc.VectorSubcoreMesh(core_axis_name="c", subcore_axis_name="s")


def _wid_base(c, s):
    return c * (E // NC) + s * E_PER_TILE


def _zero_stripe(acc_sh, rb, c, s):
    def _zero_row(r, _):
        for j in range(D // 16):
            rb[r, pl.ds(j * 16, 16)] = jnp.zeros((16,), jnp.float32)
        return 0

    lax.fori_loop(0, CHUNK, _zero_row, 0)
    stripe = s * STRIPE
    for k in range(7):
        pltpu.sync_copy(rb, acc_sh.at[pl.ds(stripe + k * CHUNK, CHUNK)])
    pltpu.sync_copy(rb.at[pl.ds(0, 64)], acc_sh.at[pl.ds(stripe + 560, 64)])

    @pl.when(s == NS - 1)
    def _zero_rem():
        pltpu.sync_copy(rb.at[pl.ds(0, REM)], acc_sh.at[pl.ds(NS * STRIPE, REM)])


def _copy_out(acc_sh, out_hbm, rb, c, s):
    stripe = s * STRIPE
    for k in range(7):
        pltpu.sync_copy(acc_sh.at[pl.ds(stripe + k * CHUNK, CHUNK)], rb)
        pltpu.sync_copy(rb, out_hbm.at[pl.ds(c * N + stripe + k * CHUNK, CHUNK)])
    pltpu.sync_copy(acc_sh.at[pl.ds(stripe + 560, 64)], rb.at[pl.ds(0, 64)])
    pltpu.sync_copy(rb.at[pl.ds(0, 64)],
                    out_hbm.at[pl.ds(c * N + stripe + 560, 64)])

    @pl.when(s == NS - 1)
    def _copy_rem():
        pltpu.sync_copy(acc_sh.at[pl.ds(NS * STRIPE, REM)], rb.at[pl.ds(0, REM)])
        pltpu.sync_copy(rb.at[pl.ds(0, REM)],
                        out_hbm.at[pl.ds(c * N + NS * STRIPE, REM)])


@functools.partial(
    pl.kernel,
    mesh=_mesh,
    out_type=jax.ShapeDtypeStruct((NC * N, D), jnp.float32),
    scratch_types=[
        pltpu.VMEM((E_PER_TILE,), jnp.int32),
        pltpu.VMEM((E_PER_TILE,), jnp.float32),
        pltpu.VMEM((CHUNK,), jnp.int32),
        pltpu.VMEM((CHUNK,), jnp.int32),
        pltpu.VMEM((CHUNK, D), jnp.float32),
        pltpu.VMEM((CHUNK, D), jnp.float32),
        pltpu.VMEM_SHARED((N, D), jnp.float32),
        pltpu.SemaphoreType.DMA,
    ],
)
def _deg_kernel(dst_hbm, ew_hbm, out_hbm, dst_all, ew_all,
                db0, db1, rb0, rb1, acc_sh, ssem):
    c = lax.axis_index("c")
    s = lax.axis_index("s")
    base = _wid_base(c, s)
    dbs = (db0, db1)
    rbs = (rb0, rb1)

    pltpu.sync_copy(dst_hbm.at[pl.ds(base, E_PER_TILE)], dst_all)
    pltpu.sync_copy(ew_hbm.at[pl.ds(base, E_PER_TILE)], ew_all)
    _zero_stripe(acc_sh, rb0, c, s)
    plsc.subcore_barrier()

    def _drain_s():
        pltpu.make_async_copy(out_hbm.at[pl.ds(0, CHUNK)], rb0, ssem).wait()

    def _build_and_scatter(i, db, rb):
        for g in range(CHUNK // 16):
            db[pl.ds(g * 16, 16)] = dst_all[pl.ds(i * CHUNK + g * 16, 16)]

        def _group(g, _):
            wv = ew_all[pl.ds(i * CHUNK + g * 16, 16)]
            for k in range(16):
                w = jnp.full((16,), wv[k], jnp.float32)
                for j in range(D // 16):
                    rb[g * 16 + k, pl.ds(j * 16, 16)] = w
            return 0

        lax.fori_loop(0, CHUNK // 16, _group, 0)
        pltpu.async_copy(rb, acc_sh.at[db], ssem, add=True)

    def _pair(sup, _):
        for b in range(2):
            i = sup * 2 + b

            @pl.when(i >= 2)
            def _():
                _drain_s()

            _build_and_scatter(i, dbs[b], rbs[b])
        return 0

    lax.fori_loop(0, NCHUNK // 2, _pair, 0)
    _drain_s()
    _drain_s()
    _build_and_scatter(NCHUNK - 1, db0, rb0)
    _drain_s()
    plsc.subcore_barrier()
    _copy_out(acc_sh, out_hbm, rb1, c, s)


@functools.partial(
    pl.kernel,
    mesh=_mesh,
    out_type=jax.ShapeDtypeStruct((NC * N, D), jnp.float32),
    scratch_types=[
        pltpu.VMEM((CHUNK,), jnp.int32),
        pltpu.VMEM((CHUNK,), jnp.int32),
        pltpu.VMEM((CHUNK,), jnp.int32),
        pltpu.VMEM((CHUNK,), jnp.int32),
        pltpu.VMEM((CHUNK,), jnp.int32),
        pltpu.VMEM((CHUNK,), jnp.int32),
        pltpu.VMEM((CHUNK,), jnp.int32),
        pltpu.VMEM((CHUNK,), jnp.int32),
        pltpu.VMEM((CHUNK,), jnp.float32),
        pltpu.VMEM((CHUNK,), jnp.float32),
        pltpu.VMEM((CHUNK,), jnp.float32),
        pltpu.VMEM((CHUNK,), jnp.float32),
        pltpu.VMEM((CHUNK, D), jnp.float32),
        pltpu.VMEM((CHUNK, D), jnp.float32),
        pltpu.VMEM((CHUNK, D), jnp.float32),
        pltpu.VMEM((CHUNK, D), jnp.float32),
        pltpu.VMEM_SHARED((N, D), jnp.float32),
        pltpu.SemaphoreType.DMA,
        pltpu.SemaphoreType.DMA,
        pltpu.SemaphoreType.DMA,
    ],
)
def _edge_kernel(xw_hbm, src_hbm, dst_hbm, ew_hbm, out_hbm,
                 sb0, sb1, sb2, sb3, db0, db1, db2, db3,
                 eb0, eb1, eb2, eb3, rb0, rb1, rb2, rb3,
                 acc_sh, isem, gsem, ssem):
    c = lax.axis_index("c")
    s = lax.axis_index("s")
    base = _wid_base(c, s)
    sbs = (sb0, sb1, sb2, sb3)
    dbs = (db0, db1, db2, db3)
    ebs = (eb0, eb1, eb2, eb3)
    rbs = (rb0, rb1, rb2, rb3)

    def _istart(i, m):
        off = base + i * CHUNK
        pltpu.async_copy(src_hbm.at[pl.ds(off, CHUNK)], sbs[m], isem)
        pltpu.async_copy(dst_hbm.at[pl.ds(off, CHUNK)], dbs[m], isem)
        pltpu.async_copy(ew_hbm.at[pl.ds(off, CHUNK)], ebs[m], isem)

    def _iwait():
        for _ in range(3):
            pltpu.make_async_copy(src_hbm.at[pl.ds(0, CHUNK)], sb0, isem).wait()

    def _gstart(b):
        pltpu.async_copy(xw_hbm.at[sbs[b]], rbs[b], gsem)

    def _gwait():
        pltpu.make_async_copy(xw_hbm.at[pl.ds(0, CHUNK)], rb0, gsem).wait()

    def _swait():
        pltpu.make_async_copy(xw_hbm.at[pl.ds(0, CHUNK)], rb0, ssem).wait()

    for i in range(3):
        off = base + i * CHUNK
        pltpu.sync_copy(src_hbm.at[pl.ds(off, CHUNK)], sbs[i])
        pltpu.sync_copy(dst_hbm.at[pl.ds(off, CHUNK)], dbs[i])
        pltpu.sync_copy(ew_hbm.at[pl.ds(off, CHUNK)], ebs[i])
    _zero_stripe(acc_sh, rb3, c, s)
    _gstart(0)
    _gstart(1)
    plsc.subcore_barrier()

    def _process(i, b):
        def _group(g, _):
            wv = ebs[b][pl.ds(g * 16, 16)]
            for k in range(16):
                w = jnp.full((16,), wv[k], jnp.float32)
                r = g * 16 + k
                for j in range(D // 16):
                    sl = pl.ds(j * 16, 16)
                    rbs[b][r, sl] = rbs[b][r, sl] * w
            return 0

        lax.fori_loop(0, CHUNK // 16, _group, 0)
        pltpu.async_copy(rbs[b], acc_sh.at[dbs[b]], ssem, add=True)

    def _quad(sup, _):
        for b in range(4):
            i = sup * 4 + b

            @pl.when(i + 2 <= NCHUNK - 1)
            def _():
                @pl.when(i >= 1)
                def _():
                    _iwait()

                _gstart((b + 2) % 4)

            _gwait()

            @pl.when(i >= 1)
            def _():
                _swait()

            @pl.when(i + 3 <= NCHUNK - 1)
            def _():
                _istart(i + 3, (b + 3) % 4)

            _process(i, b)
        return 0

    lax.fori_loop(0, NCHUNK // 4, _quad, 0)
    _swait()
    _gwait()
    _process(NCHUNK - 1, 0)
    _swait()
    plsc.subcore_barrier()
    _copy_out(acc_sh, out_hbm, rb1, c, s)


ROWS_TC = 400
GRID = N // ROWS_TC


def _dis_block(d0, d1):
    deg = 1.0 + d0[:, 0:1] + d1[:, 0:1]
    return jnp.where(deg > 0, lax.rsqrt(jnp.maximum(deg, 1e-12)), 0.0)


def _m1a_body(x_ref, w_ref, o_ref):
    o_ref[...] = jnp.dot(x_ref[...], w_ref[...],
                         preferred_element_type=jnp.float32,
                         precision=lax.Precision.HIGHEST)


def _m1b_body(xw_ref, d0_ref, d1_ref, o_ref):
    dis = _dis_block(d0_ref[...], d1_ref[...])
    o_ref[...] = xw_ref[...] * dis


def _fm_body(a0_ref, a1_ref, xwp_ref, d0_ref, d1_ref, b_ref, w_ref, o_ref):
    dis = _dis_block(d0_ref[...], d1_ref[...])
    h = dis * (a0_ref[...] + a1_ref[...] + xwp_ref[...]) + b_ref[...]
    h = jnp.maximum(h, 0.0)
    hw = jnp.dot(h, w_ref[...], preferred_element_type=jnp.float32,
                 precision=lax.Precision.HIGHEST)
    o_ref[...] = hw * dis


def _f2_body(a0_ref, a1_ref, xwp_ref, d0_ref, d1_ref, b_ref, o_ref):
    dis = _dis_block(d0_ref[...], d1_ref[...])
    o_ref[...] = dis * (a0_ref[...] + a1_ref[...] + xwp_ref[...]) + b_ref[...]


_row_spec = pl.BlockSpec((ROWS_TC, D), lambda i: (i, 0))
_deg_spec = pl.BlockSpec((ROWS_TC, 8), lambda i: (i, 0))
_w_spec = pl.BlockSpec((D, D), lambda i: (0, 0))
_b_spec = pl.BlockSpec((1, D), lambda i: (0, 0))
_out_struct = jax.ShapeDtypeStruct((N, D), jnp.float32)

_m1a = pl.pallas_call(
    _m1a_body, grid=(GRID,),
    in_specs=[_row_spec, _w_spec],
    out_specs=_row_spec, out_shape=_out_struct)

_m1b = pl.pallas_call(
    _m1b_body, grid=(GRID,),
    in_specs=[_row_spec, _deg_spec, _deg_spec],
    out_specs=_row_spec, out_shape=_out_struct)

_fm = pl.pallas_call(
    _fm_body, grid=(GRID,),
    in_specs=[_row_spec, _row_spec, _row_spec, _deg_spec, _deg_spec,
              _b_spec, _w_spec],
    out_specs=_row_spec, out_shape=_out_struct)

_f2 = pl.pallas_call(
    _f2_body, grid=(GRID,),
    in_specs=[_row_spec, _row_spec, _row_spec, _deg_spec, _deg_spec, _b_spec],
    out_specs=_row_spec, out_shape=_out_struct)


def kernel(x, edge_index, edge_weight, W1, b1, W2, b2):
    src = edge_index[0]
    dst = edge_index[1]

    xw1 = _m1a(x, W1)
    deg_parts = _deg_kernel(dst, edge_weight)
    d0, d1 = deg_parts[:N, :8], deg_parts[N:, :8]

    xw1p = _m1b(xw1, d0, d1)
    a1 = _edge_kernel(xw1p, src, dst, edge_weight)
    xw2p = _fm(a1[:N], a1[N:], xw1p, d0, d1, b1.reshape(1, D), W2)
    a2 = _edge_kernel(xw2p, src, dst, edge_weight)
    out = _f2(a2[:N], a2[N:], xw2p, d0, d1, b2.reshape(1, D))
    return out


# speedup vs baseline: 1.0620x; 1.0498x over previous
import functools

import jax
import jax.numpy as jnp
from jax import lax
from jax.experimental import pallas as pl
from jax.experimental.pallas import tpu as pltpu
from jax.experimental.pallas import tpu_sc as plsc

N = 10000
E = 320000
D = 128

NC = 2
NS = 16
L = 16
NW = NC * NS

E_PER_TILE = E // NW
CHUNK = 80
NCHUNK = E_PER_TILE // CHUNK
STRIPE = 624
REM = N - NS * STRIPE

_mesh = plsc.VectorSubcoreMesh(core_axis_name="c", subcore_axis_name="s")


def _wid_base(c, s):
    return c * (E // NC) + s * E_PER_TILE


def _zero_stripe(acc_sh, rb, c, s):
    def _zero_row(r, _):
        for j in range(D // 16):
            rb[r, pl.ds(j * 16, 16)] = jnp.zeros((16,), jnp.float32)
        return 0

    lax.fori_loop(0, CHUNK, _zero_row, 0)
    stripe = s * STRIPE
    for k in range(7):
        pltpu.sync_copy(rb, acc_sh.at[pl.ds(stripe + k * CHUNK, CHUNK)])
    pltpu.sync_copy(rb.at[pl.ds(0, 64)], acc_sh.at[pl.ds(stripe + 560, 64)])

    @pl.when(s == NS - 1)
    def _zero_rem():
        pltpu.sync_copy(rb.at[pl.ds(0, REM)], acc_sh.at[pl.ds(NS * STRIPE, REM)])


def _copy_out(acc_sh, out_hbm, rb, c, s):
    stripe = s * STRIPE
    for k in range(7):
        pltpu.sync_copy(acc_sh.at[pl.ds(stripe + k * CHUNK, CHUNK)], rb)
        pltpu.sync_copy(rb, out_hbm.at[pl.ds(c * N + stripe + k * CHUNK, CHUNK)])
    pltpu.sync_copy(acc_sh.at[pl.ds(stripe + 560, 64)], rb.at[pl.ds(0, 64)])
    pltpu.sync_copy(rb.at[pl.ds(0, 64)],
                    out_hbm.at[pl.ds(c * N + stripe + 560, 64)])

    @pl.when(s == NS - 1)
    def _copy_rem():
        pltpu.sync_copy(acc_sh.at[pl.ds(NS * STRIPE, REM)], rb.at[pl.ds(0, REM)])
        pltpu.sync_copy(rb.at[pl.ds(0, REM)],
                        out_hbm.at[pl.ds(c * N + NS * STRIPE, REM)])


@functools.partial(
    pl.kernel,
    mesh=_mesh,
    out_type=jax.ShapeDtypeStruct((NC * N, D), jnp.float32),
    scratch_types=[
        pltpu.VMEM((E_PER_TILE,), jnp.int32),
        pltpu.VMEM((E_PER_TILE,), jnp.float32),
        pltpu.VMEM((CHUNK,), jnp.int32),
        pltpu.VMEM((CHUNK,), jnp.int32),
        pltpu.VMEM((CHUNK, D), jnp.float32),
        pltpu.VMEM((CHUNK, D), jnp.float32),
        pltpu.VMEM_SHARED((N, D), jnp.float32),
        pltpu.SemaphoreType.DMA,
    ],
)
def _deg_kernel(dst_hbm, ew_hbm, out_hbm, dst_all, ew_all,
                db0, db1, rb0, rb1, acc_sh, ssem):
    c = lax.axis_index("c")
    s = lax.axis_index("s")
    base = _wid_base(c, s)
    dbs = (db0, db1)
    rbs = (rb0, rb1)

    pltpu.sync_copy(dst_hbm.at[pl.ds(base, E_PER_TILE)], dst_all)
    pltpu.sync_copy(ew_hbm.at[pl.ds(base, E_PER_TILE)], ew_all)
    _zero_stripe(acc_sh, rb0, c, s)
    plsc.subcore_barrier()

    def _drain_s():
        pltpu.make_async_copy(out_hbm.at[pl.ds(0, CHUNK)], rb0, ssem).wait()

    def _build_and_scatter(i, db, rb):
        for g in range(CHUNK // 16):
            db[pl.ds(g * 16, 16)] = dst_all[pl.ds(i * CHUNK + g * 16, 16)]

        def _group(g, _):
            wv = ew_all[pl.ds(i * CHUNK + g * 16, 16)]
            for k in range(16):
                w = jnp.full((16,), wv[k], jnp.float32)
                for j in range(D // 16):
                    rb[g * 16 + k, pl.ds(j * 16, 16)] = w
            return 0

        lax.fori_loop(0, CHUNK // 16, _group, 0)
        pltpu.async_copy(rb, acc_sh.at[db], ssem, add=True)

    def _pair(sup, _):
        for b in range(2):
            i = sup * 2 + b

            @pl.when(i >= 2)
            def _():
                _drain_s()

            _build_and_scatter(i, dbs[b], rbs[b])
        return 0

    lax.fori_loop(0, NCHUNK // 2, _pair, 0)
    _drain_s()
    _drain_s()
    _build_and_scatter(NCHUNK - 1, db0, rb0)
    _drain_s()
    plsc.subcore_barrier()
    _copy_out(acc_sh, out_hbm, rb1, c, s)


@functools.partial(
    pl.kernel,
    mesh=_mesh,
    out_type=jax.ShapeDtypeStruct((NC * N, D), jnp.float32),
    scratch_types=[
        pltpu.VMEM((CHUNK,), jnp.int32),
        pltpu.VMEM((CHUNK,), jnp.int32),
        pltpu.VMEM((CHUNK,), jnp.int32),
        pltpu.VMEM((CHUNK,), jnp.int32),
        pltpu.VMEM((CHUNK,), jnp.int32),
        pltpu.VMEM((CHUNK,), jnp.int32),
        pltpu.VMEM((CHUNK,), jnp.int32),
        pltpu.VMEM((CHUNK,), jnp.int32),
        pltpu.VMEM((CHUNK,), jnp.float32),
        pltpu.VMEM((CHUNK,), jnp.float32),
        pltpu.VMEM((CHUNK,), jnp.float32),
        pltpu.VMEM((CHUNK,), jnp.float32),
        pltpu.VMEM((CHUNK, D), jnp.float32),
        pltpu.VMEM((CHUNK, D), jnp.float32),
        pltpu.VMEM((CHUNK, D), jnp.float32),
        pltpu.VMEM((CHUNK, D), jnp.float32),
        pltpu.VMEM_SHARED((N, D), jnp.float32),
        pltpu.SemaphoreType.DMA,
        pltpu.SemaphoreType.DMA,
        pltpu.SemaphoreType.DMA,
    ],
)
def _edge_kernel(xw_hbm, src_hbm, dst_hbm, ew_hbm, out_hbm,
                 sb0, sb1, sb2, sb3, db0, db1, db2, db3,
                 eb0, eb1, eb2, eb3, rb0, rb1, rb2, rb3,
                 acc_sh, isem, gsem, ssem):
    c = lax.axis_index("c")
    s = lax.axis_index("s")
    base = _wid_base(c, s)
    sbs = (sb0, sb1, sb2, sb3)
    dbs = (db0, db1, db2, db3)
    ebs = (eb0, eb1, eb2, eb3)
    rbs = (rb0, rb1, rb2, rb3)

    def _istart(i, m):
        off = base + i * CHUNK
        pltpu.async_copy(src_hbm.at[pl.ds(off, CHUNK)], sbs[m], isem)
        pltpu.async_copy(dst_hbm.at[pl.ds(off, CHUNK)], dbs[m], isem)
        pltpu.async_copy(ew_hbm.at[pl.ds(off, CHUNK)], ebs[m], isem)

    def _iwait():
        for _ in range(3):
            pltpu.make_async_copy(src_hbm.at[pl.ds(0, CHUNK)], sb0, isem).wait()

    def _gstart(b):
        pltpu.async_copy(xw_hbm.at[sbs[b]], rbs[b], gsem)

    def _gwait():
        pltpu.make_async_copy(xw_hbm.at[pl.ds(0, CHUNK)], rb0, gsem).wait()

    def _swait():
        pltpu.make_async_copy(xw_hbm.at[pl.ds(0, CHUNK)], rb0, ssem).wait()

    for i in range(3):
        off = base + i * CHUNK
        pltpu.sync_copy(src_hbm.at[pl.ds(off, CHUNK)], sbs[i])
        pltpu.sync_copy(dst_hbm.at[pl.ds(off, CHUNK)], dbs[i])
        pltpu.sync_copy(ew_hbm.at[pl.ds(off, CHUNK)], ebs[i])
    _zero_stripe(acc_sh, rb3, c, s)
    _gstart(0)
    _gstart(1)
    plsc.subcore_barrier()

    def _process(i, b):
        def _group(g, _):
            wv = ebs[b][pl.ds(g * 16, 16)]
            for k in range(16):
                w = jnp.full((16,), wv[k], jnp.float32)
                r = g * 16 + k
                for j in range(D // 16):
                    sl = pl.ds(j * 16, 16)
                    rbs[b][r, sl] = rbs[b][r, sl] * w
            return 0

        lax.fori_loop(0, CHUNK // 16, _group, 0)
        pltpu.async_copy(rbs[b], acc_sh.at[dbs[b]], ssem, add=True)

    def _quad(sup, _):
        for b in range(4):
            i = sup * 4 + b

            @pl.when(i + 2 <= NCHUNK - 1)
            def _():
                @pl.when(i >= 1)
                def _():
                    _iwait()

                _gstart((b + 2) % 4)

            _gwait()

            @pl.when(i >= 1)
            def _():
                _swait()

            @pl.when(i + 3 <= NCHUNK - 1)
            def _():
                _istart(i + 3, (b + 3) % 4)

            _process(i, b)
        return 0

    lax.fori_loop(0, NCHUNK // 4, _quad, 0)
    _swait()
    _gwait()
    _process(NCHUNK - 1, 0)
    _swait()
    plsc.subcore_barrier()
    _copy_out(acc_sh, out_hbm, rb1, c, s)


ROWS_TC = 400
GRID = N // ROWS_TC


def _dis_block(d0, d1):
    deg = 1.0 + d0[:, 0:1] + d1[:, 0:1]
    return jnp.where(deg > 0, lax.rsqrt(jnp.maximum(deg, 1e-12)), 0.0)


def _m1a_body(x_ref, w_ref, o_ref):
    o_ref[...] = jnp.dot(x_ref[...], w_ref[...],
                         preferred_element_type=jnp.float32,
                         precision=lax.Precision.HIGHEST)


def _m1b_body(xw_ref, d0_ref, d1_ref, o_ref):
    dis = _dis_block(d0_ref[...], d1_ref[...])
    o_ref[...] = xw_ref[...] * dis


def _fm_body(a0_ref, a1_ref, xwp_ref, d0_ref, d1_ref, b_ref, w_ref, o_ref):
    dis = _dis_block(d0_ref[...], d1_ref[...])
    h = dis * (a0_ref[...] + a1_ref[...] + xwp_ref[...]) + b_ref[...]
    h = jnp.maximum(h, 0.0)
    hw = jnp.dot(h, w_ref[...], preferred_element_type=jnp.float32,
                 precision=lax.Precision.HIGHEST)
    o_ref[...] = hw * dis


def _f2_body(a0_ref, a1_ref, xwp_ref, d0_ref, d1_ref, b_ref, o_ref):
    dis = _dis_block(d0_ref[...], d1_ref[...])
    o_ref[...] = dis * (a0_ref[...] + a1_ref[...] + xwp_ref[...]) + b_ref[...]


_row_spec = pl.BlockSpec((ROWS_TC, D), lambda i: (i, 0))
_acc0_spec = pl.BlockSpec((ROWS_TC, D), lambda i: (i, 0))
_acc1_spec = pl.BlockSpec((ROWS_TC, D), lambda i: (i + GRID, 0))
_deg0_spec = pl.BlockSpec((ROWS_TC, D), lambda i: (i, 0))
_deg1_spec = pl.BlockSpec((ROWS_TC, D), lambda i: (i + GRID, 0))
_w_spec = pl.BlockSpec((D, D), lambda i: (0, 0))
_b_spec = pl.BlockSpec((1, D), lambda i: (0, 0))
_out_struct = jax.ShapeDtypeStruct((N, D), jnp.float32)

_m1a = pl.pallas_call(
    _m1a_body, grid=(GRID,),
    in_specs=[_row_spec, _w_spec],
    out_specs=_row_spec, out_shape=_out_struct)

_m1b = pl.pallas_call(
    _m1b_body, grid=(GRID,),
    in_specs=[_row_spec, _deg0_spec, _deg1_spec],
    out_specs=_row_spec, out_shape=_out_struct)

_fm = pl.pallas_call(
    _fm_body, grid=(GRID,),
    in_specs=[_acc0_spec, _acc1_spec, _row_spec, _deg0_spec, _deg1_spec,
              _b_spec, _w_spec],
    out_specs=_row_spec, out_shape=_out_struct)

_f2 = pl.pallas_call(
    _f2_body, grid=(GRID,),
    in_specs=[_acc0_spec, _acc1_spec, _row_spec, _deg0_spec, _deg1_spec,
              _b_spec],
    out_specs=_row_spec, out_shape=_out_struct)


def kernel(x, edge_index, edge_weight, W1, b1, W2, b2):
    src = edge_index[0]
    dst = edge_index[1]

    xw1 = _m1a(x, W1)
    deg_parts = _deg_kernel(dst, edge_weight)

    xw1p = _m1b(xw1, deg_parts, deg_parts)
    a1 = _edge_kernel(xw1p, src, dst, edge_weight)
    xw2p = _fm(a1, a1, xw1p, deg_parts, deg_parts, b1.reshape(1, D), W2)
    a2 = _edge_kernel(xw2p, src, dst, edge_weight)
    out = _f2(a2, a2, xw2p, deg_parts, deg_parts, b2.reshape(1, D))
    return out
